# baseline (device time: 69005 ns/iter reference)
import jax
import jax.numpy as jnp
from jax import lax
from jax.experimental import pallas as pl
from jax.experimental.pallas import tpu as pltpu

B, S, H, D = 2, 512, 8, 64
BH = B * H
SCALE = D ** -0.5

_CompilerParams = getattr(pltpu, "CompilerParams", None) or pltpu.TPUCompilerParams


def kernel(Q, K, V):
    f32 = jnp.float32
    bf16 = jnp.bfloat16
    Qp = jnp.transpose(Q, (0, 2, 1, 3)).reshape(BH, S, D).astype(bf16)
    Kp = jnp.transpose(K, (0, 2, 1, 3)).reshape(BH, S, D).astype(bf16)
    Vp = jnp.transpose(V, (0, 2, 1, 3)).reshape(BH, S, D).astype(bf16)

    def body(q_ref, k_ref, v_ref, o_ref, krem_ref, vrem_ref, send_sems, recv_sems):
        my_x = lax.axis_index("x")
        my_y = lax.axis_index("y")
        my_z = lax.axis_index("z")
        peer = (my_x, 1 - my_y, my_z)

        barrier = pltpu.get_barrier_semaphore()
        pl.semaphore_signal(
            barrier, inc=1, device_id=peer, device_id_type=pl.DeviceIdType.MESH
        )
        pl.semaphore_wait(barrier, 1)

        rdma_k = pltpu.make_async_remote_copy(
            src_ref=k_ref,
            dst_ref=krem_ref,
            send_sem=send_sems.at[0],
            recv_sem=recv_sems.at[0],
            device_id=peer,
            device_id_type=pl.DeviceIdType.MESH,
        )
        rdma_v = pltpu.make_async_remote_copy(
            src_ref=v_ref,
            dst_ref=vrem_ref,
            send_sem=send_sems.at[1],
            recv_sem=recv_sems.at[1],
            device_id=peer,
            device_id_type=pl.DeviceIdType.MESH,
        )
        rdma_k.start()
        rdma_v.start()
        rdma_k.wait()
        rdma_v.wait()

        for i in range(BH):
            q = q_ref[i]
            s_loc = lax.dot_general(
                q, k_ref[i], (((1,), (1,)), ((), ())), preferred_element_type=f32
            ) * SCALE
            s_rem = lax.dot_general(
                q, krem_ref[i], (((1,), (1,)), ((), ())), preferred_element_type=f32
            ) * SCALE
            m = jnp.maximum(
                jnp.max(s_loc, axis=1, keepdims=True),
                jnp.max(s_rem, axis=1, keepdims=True),
            )
            p_loc = jnp.exp(s_loc - m)
            p_rem = jnp.exp(s_rem - m)
            l = jnp.sum(p_loc, axis=1, keepdims=True) + jnp.sum(
                p_rem, axis=1, keepdims=True
            )
            o = lax.dot_general(
                p_loc.astype(bf16),
                v_ref[i],
                (((1,), (0,)), ((), ())),
                preferred_element_type=f32,
            )
            o = o + lax.dot_general(
                p_rem.astype(bf16),
                vrem_ref[i],
                (((1,), (0,)), ((), ())),
                preferred_element_type=f32,
            )
            o_ref[i] = o / l


    out = pl.pallas_call(
        body,
        out_shape=jax.ShapeDtypeStruct((BH, S, D), f32),
        in_specs=[pl.BlockSpec(memory_space=pltpu.VMEM)] * 3,
        out_specs=pl.BlockSpec(memory_space=pltpu.VMEM),
        scratch_shapes=[
            pltpu.VMEM((BH, S, D), bf16),
            pltpu.VMEM((BH, S, D), bf16),
            pltpu.SemaphoreType.DMA((2,)),
            pltpu.SemaphoreType.DMA((2,)),
        ],
        compiler_params=_CompilerParams(collective_id=0),
    )(Qp, Kp, Vp)

    return out.reshape(B, H, S, D).transpose(0, 2, 1, 3)


# device time: 57368 ns/iter; 1.2028x vs baseline; 1.2028x over previous
import os

import jax
import jax.numpy as jnp
from jax import lax
from jax.experimental import pallas as pl
from jax.experimental.pallas import tpu as pltpu

_MODE = os.environ.get("KMODE", "full")

B, S, H, D = 2, 512, 8, 64
BH = B * H
SCALE = D ** -0.5

_CompilerParams = getattr(pltpu, "CompilerParams", None) or pltpu.TPUCompilerParams


def kernel(Q, K, V):
    f32 = jnp.float32
    bf16 = jnp.bfloat16
    Qp = jnp.transpose(Q, (0, 2, 1, 3)).reshape(BH, S, D).astype(bf16)
    Kp = jnp.transpose(K, (0, 2, 1, 3)).reshape(BH, S, D).astype(bf16)
    Vp = jnp.transpose(V, (0, 2, 1, 3)).reshape(BH, S, D).astype(bf16)

    def body(q_ref, k_ref, v_ref, o_ref, krem_ref, vrem_ref, send_sems, recv_sems):
        my_x = lax.axis_index("x")
        my_y = lax.axis_index("y")
        my_z = lax.axis_index("z")
        peer = (my_x, 1 - my_y, my_z)

        barrier = pltpu.get_barrier_semaphore()
        pl.semaphore_signal(
            barrier, inc=1, device_id=peer, device_id_type=pl.DeviceIdType.MESH
        )
        pl.semaphore_wait(barrier, 1)

        rdma_k = pltpu.make_async_remote_copy(
            src_ref=k_ref,
            dst_ref=krem_ref,
            send_sem=send_sems.at[0],
            recv_sem=recv_sems.at[0],
            device_id=peer,
            device_id_type=pl.DeviceIdType.MESH,
        )
        rdma_v = pltpu.make_async_remote_copy(
            src_ref=v_ref,
            dst_ref=vrem_ref,
            send_sem=send_sems.at[1],
            recv_sem=recv_sems.at[1],
            device_id=peer,
            device_id_type=pl.DeviceIdType.MESH,
        )
        if _MODE != "compute":
            rdma_k.start()
            rdma_v.start()
            rdma_k.wait()
            rdma_v.wait()

        if _MODE == "comm":
            for i in range(BH):
                o_ref[i] = krem_ref[i].astype(f32)
            return
        if _MODE == "compute":
            krem_ref, vrem_ref = k_ref, v_ref

        for i in range(BH):
            q = q_ref[i]
            s_loc = lax.dot_general(
                q, k_ref[i], (((1,), (1,)), ((), ())), preferred_element_type=f32
            ) * SCALE
            s_rem = lax.dot_general(
                q, krem_ref[i], (((1,), (1,)), ((), ())), preferred_element_type=f32
            ) * SCALE
            m = jnp.maximum(
                jnp.max(s_loc, axis=1, keepdims=True),
                jnp.max(s_rem, axis=1, keepdims=True),
            )
            p_loc = jnp.exp(s_loc - m)
            p_rem = jnp.exp(s_rem - m)
            l = jnp.sum(p_loc, axis=1, keepdims=True) + jnp.sum(
                p_rem, axis=1, keepdims=True
            )
            o = lax.dot_general(
                p_loc.astype(bf16),
                v_ref[i],
                (((1,), (0,)), ((), ())),
                preferred_element_type=f32,
            )
            o = o + lax.dot_general(
                p_rem.astype(bf16),
                vrem_ref[i],
                (((1,), (0,)), ((), ())),
                preferred_element_type=f32,
            )
            o_ref[i] = o / l


    out = pl.pallas_call(
        body,
        out_shape=jax.ShapeDtypeStruct((BH, S, D), f32),
        in_specs=[pl.BlockSpec(memory_space=pltpu.VMEM)] * 3,
        out_specs=pl.BlockSpec(memory_space=pltpu.VMEM),
        scratch_shapes=[
            pltpu.VMEM((BH, S, D), bf16),
            pltpu.VMEM((BH, S, D), bf16),
            pltpu.SemaphoreType.DMA((2,)),
            pltpu.SemaphoreType.DMA((2,)),
        ],
        compiler_params=_CompilerParams(collective_id=0),
    )(Qp, Kp, Vp)

    return out.reshape(B, H, S, D).transpose(0, 2, 1, 3)


# device time: 39577 ns/iter; 1.7436x vs baseline; 1.4495x over previous
import os

import jax
import jax.numpy as jnp
from jax import lax
from jax.experimental import pallas as pl
from jax.experimental.pallas import tpu as pltpu

_MODE = os.environ.get("KMODE", "full")

B, S, H, D = 2, 512, 8, 64
BH = B * H
SCALE = D ** -0.5
NCHUNK = 8
CHUNK = BH // NCHUNK

_CompilerParams = getattr(pltpu, "CompilerParams", None) or pltpu.TPUCompilerParams


def kernel(Q, K, V):
    f32 = jnp.float32
    bf16 = jnp.bfloat16
    Qp = jnp.transpose(Q, (0, 2, 1, 3)).reshape(BH, S, D).astype(bf16)
    Kp = jnp.transpose(K, (0, 2, 1, 3)).reshape(BH, S, D).astype(bf16)
    Vp = jnp.transpose(V, (0, 2, 1, 3)).reshape(BH, S, D).astype(bf16)
    KVp = jnp.concatenate([Kp, Vp], axis=-1)
    Qw = jnp.concatenate([Qp, jnp.zeros_like(Qp)], axis=-1)

    def body(q_ref, kv_ref, o_ref, kvrem_ref, l_ref, send_sems, recv_sems):
        my_x = lax.axis_index("x")
        my_y = lax.axis_index("y")
        my_z = lax.axis_index("z")
        peer = (my_x, 1 - my_y, my_z)

        barrier = pltpu.get_barrier_semaphore()
        pl.semaphore_signal(
            barrier, inc=1, device_id=peer, device_id_type=pl.DeviceIdType.MESH
        )
        pl.semaphore_wait(barrier, 1)

        def make_rdma(c):
            sl = pl.ds(c * CHUNK, CHUNK)
            return pltpu.make_async_remote_copy(
                src_ref=kv_ref.at[sl],
                dst_ref=kvrem_ref.at[sl],
                send_sem=send_sems.at[c],
                recv_sem=recv_sems.at[c],
                device_id=peer,
                device_id_type=pl.DeviceIdType.MESH,
            )

        rdmas = [make_rdma(c) for c in range(NCHUNK)]
        if _MODE != "compute":
            for r in rdmas:
                r.start()

        dn_qk = (((1,), (1,)), ((), ()))
        dn_pv = (((1,), (0,)), ((), ()))

        def attn_block(i, kv, first):
            kvi = kv[i]
            s_blk = (
                lax.dot_general(q_ref[i], kvi, dn_qk, preferred_element_type=f32)
                * SCALE
            )
            p = jnp.exp(s_blk)
            l_blk = jnp.sum(p, axis=1, keepdims=True)
            o_blk = lax.dot_general(
                p.astype(bf16), kvi, dn_pv, preferred_element_type=f32
            )
            if first:
                l_ref[i] = l_blk
                o_ref[i] = o_blk
            else:
                o_ref[i] = (o_ref[i] + o_blk) / (l_ref[i] + l_blk)

        if _MODE != "comm":
            for i in range(BH):
                attn_block(i, kv_ref, first=True)

        if _MODE == "compute":
            for i in range(BH):
                attn_block(i, kv_ref, first=False)
            return

        for c in range(NCHUNK):
            rdmas[c].wait_recv()
            if _MODE == "comm":
                continue
            for i in range(c * CHUNK, (c + 1) * CHUNK):
                attn_block(i, kvrem_ref, first=False)

        if _MODE == "comm":
            for i in range(BH):
                o_ref[i] = kvrem_ref[i].astype(f32)

        for r in rdmas:
            r.wait_send()


    out = pl.pallas_call(
        body,
        out_shape=jax.ShapeDtypeStruct((BH, S, 2 * D), f32),
        in_specs=[pl.BlockSpec(memory_space=pltpu.VMEM)] * 2,
        out_specs=pl.BlockSpec(memory_space=pltpu.VMEM),
        scratch_shapes=[
            pltpu.VMEM((BH, S, 2 * D), bf16),
            pltpu.VMEM((BH, S, 1), f32),
            pltpu.SemaphoreType.DMA((NCHUNK,)),
            pltpu.SemaphoreType.DMA((NCHUNK,)),
        ],
        compiler_params=_CompilerParams(collective_id=0),
    )(Qw, KVp)

    return out[:, :, D:].reshape(B, H, S, D).transpose(0, 2, 1, 3)


# device time: 36514 ns/iter; 1.8898x vs baseline; 1.0839x over previous
import os

import jax
import jax.numpy as jnp
from jax import lax
from jax.experimental import pallas as pl
from jax.experimental.pallas import tpu as pltpu

_MODE = os.environ.get("KMODE", "full")

B, S, H, D = 2, 512, 8, 64
BH = B * H
SCALE = D ** -0.5
NCHUNK = 16
CHUNK = BH // NCHUNK

_CompilerParams = getattr(pltpu, "CompilerParams", None) or pltpu.TPUCompilerParams


def kernel(Q, K, V):
    f32 = jnp.float32
    bf16 = jnp.bfloat16
    Qp = jnp.transpose(Q, (0, 2, 1, 3)).reshape(BH, S, D).astype(bf16)
    KVp = (
        jnp.concatenate([K, V], axis=-1)
        .astype(bf16)
        .transpose(0, 2, 1, 3)
        .reshape(BH, S, 2 * D)
    )

    def body(q_ref, kv_ref, o_ref, kvrem_ref, l_ref, send_sems, recv_sems):
        my_x = lax.axis_index("x")
        my_y = lax.axis_index("y")
        my_z = lax.axis_index("z")
        peer = (my_x, 1 - my_y, my_z)

        barrier = pltpu.get_barrier_semaphore()
        pl.semaphore_signal(
            barrier, inc=1, device_id=peer, device_id_type=pl.DeviceIdType.MESH
        )
        pl.semaphore_wait(barrier, 1)

        def make_rdma(c):
            sl = pl.ds(c * CHUNK, CHUNK)
            return pltpu.make_async_remote_copy(
                src_ref=kv_ref.at[sl],
                dst_ref=kvrem_ref.at[sl],
                send_sem=send_sems.at[c],
                recv_sem=recv_sems.at[c],
                device_id=peer,
                device_id_type=pl.DeviceIdType.MESH,
            )

        rdmas = [make_rdma(c) for c in range(NCHUNK)]
        if _MODE != "compute":
            for r in rdmas:
                r.start()

        dn_qk = (((1,), (1,)), ((), ()))
        dn_pv = (((1,), (0,)), ((), ()))
        eye_pad = jnp.eye(D, 2 * D, dtype=bf16)

        def attn_block(i, kv, first):
            kvi = kv[i]
            qw = lax.dot_general(
                q_ref[i], eye_pad, (((1,), (0,)), ((), ())), preferred_element_type=f32
            ).astype(bf16)
            s_blk = (
                lax.dot_general(qw, kvi, dn_qk, preferred_element_type=f32) * SCALE
            )
            p = jnp.exp(s_blk)
            l_blk = jnp.sum(p, axis=1, keepdims=True)
            o_blk = lax.dot_general(
                p.astype(bf16), kvi, dn_pv, preferred_element_type=f32
            )
            if first:
                l_ref[i] = l_blk
                o_ref[i] = o_blk.astype(bf16)
            else:
                o_ref[i] = (
                    (o_ref[i].astype(f32) + o_blk) / (l_ref[i] + l_blk)
                ).astype(bf16)

        if _MODE != "comm":
            for i in range(BH):
                attn_block(i, kv_ref, first=True)

        if _MODE == "compute":
            for i in range(BH):
                attn_block(i, kv_ref, first=False)
            return

        for c in range(NCHUNK):
            rdmas[c].wait_recv()
            if _MODE == "comm":
                continue
            for i in range(c * CHUNK, (c + 1) * CHUNK):
                attn_block(i, kvrem_ref, first=False)

        if _MODE == "comm":
            for i in range(BH):
                o_ref[i] = kvrem_ref[i]

        for r in rdmas:
            r.wait_send()


    out = pl.pallas_call(
        body,
        out_shape=jax.ShapeDtypeStruct((BH, S, 2 * D), bf16),
        in_specs=[pl.BlockSpec(memory_space=pltpu.VMEM)] * 2,
        out_specs=pl.BlockSpec(memory_space=pltpu.VMEM),
        scratch_shapes=[
            pltpu.VMEM((BH, S, 2 * D), bf16),
            pltpu.VMEM((BH, S, 1), f32),
            pltpu.SemaphoreType.DMA((NCHUNK,)),
            pltpu.SemaphoreType.DMA((NCHUNK,)),
        ],
        compiler_params=_CompilerParams(collective_id=0),
    )(Qp, KVp)

    return out[:, :, D:].reshape(B, H, S, D).transpose(0, 2, 1, 3).astype(f32)
